# baseline (speedup 1.0000x reference)
import functools

import jax
import jax.numpy as jnp
from jax import lax
from jax.experimental import pallas as pl
from jax.experimental.pallas import tpu as pltpu
from jax.experimental.pallas import tpu_sc as plsc

N = 10000
E = 320000
D = 128

NC = 2
NS = 16
NW = NC * NS
CHUNK = 128
CJ = 80
ET = CJ * CHUNK
EP = NW * ET
NP = 10112
NDUM = NP - N
RPT = NP // NS

_mesh = plsc.VectorSubcoreMesh(core_axis_name="c", subcore_axis_name="s")


@functools.partial(
    pl.kernel,
    out_type=(
        jax.ShapeDtypeStruct((NW, CJ, CHUNK), jnp.int32),
        jax.ShapeDtypeStruct((NW, CJ, CHUNK), jnp.int32),
        jax.ShapeDtypeStruct((NC, NP, D), jnp.float32),
    ),
    mesh=_mesh,
    scratch_types=[
        pltpu.VMEM((CJ, CHUNK), jnp.int32),
        pltpu.VMEM((CJ, CHUNK), jnp.int32),
        pltpu.VMEM((CHUNK, D), jnp.float32),
        pltpu.VMEM_SHARED((NP, D), jnp.float32),
        pltpu.SemaphoreType.DMA,
    ],
)
def _edge_prep(row3, col3, zrow, ones128, rowp_o, colp_o, degp_o,
               r_buf, c_buf, ones_v, deg_sh, ssem):
    c = lax.axis_index("c")
    s = lax.axis_index("s")
    wid = c * NS + s
    rem = RPT - 4 * CHUNK

    pltpu.sync_copy(zrow, ones_v)
    for p in range(4):
        pltpu.sync_copy(ones_v, deg_sh.at[pl.ds(s * RPT + p * CHUNK, CHUNK)])
    pltpu.sync_copy(ones_v.at[pl.ds(0, rem)],
                    deg_sh.at[pl.ds(s * RPT + 4 * CHUNK, rem)])
    pltpu.sync_copy(ones128, ones_v)
    pltpu.sync_copy(row3.at[wid], r_buf)
    pltpu.sync_copy(col3.at[wid], c_buf)

    base = wid * ET
    lane = lax.broadcasted_iota(jnp.int32, (16,), 0)

    def compute_row(j, _):
        for k in range(CHUNK // 16):
            r = r_buf[j, pl.ds(k * 16, 16)]
            cc = c_buf[j, pl.ds(k * 16, 16)]
            m = r == cc
            pos = base + j * CHUNK + k * 16 + lane
            dummy = N + lax.rem(pos, NDUM)
            r_buf[j, pl.ds(k * 16, 16)] = jnp.where(m, dummy, r)
            c_buf[j, pl.ds(k * 16, 16)] = jnp.where(
                m, lax.bitwise_and(pos, 8191), cc)
        return 0

    lax.fori_loop(0, CJ, compute_row, 0)
    plsc.subcore_barrier()

    def deg_row(j, _):
        pltpu.async_copy(ones_v, deg_sh.at[r_buf.at[j]], ssem, add=True)
        return 0

    lax.fori_loop(0, CJ, deg_row, 0)

    def deg_drain(j, _):
        pltpu.make_async_copy(ones_v, deg_sh.at[r_buf.at[j]], ssem).wait()
        return 0

    lax.fori_loop(0, CJ, deg_drain, 0)

    pltpu.sync_copy(r_buf, rowp_o.at[wid])
    pltpu.sync_copy(c_buf, colp_o.at[wid])
    plsc.subcore_barrier()
    for p in range(4):
        pltpu.sync_copy(deg_sh.at[pl.ds(s * RPT + p * CHUNK, CHUNK)], ones_v)
        pltpu.sync_copy(ones_v, degp_o.at[c, pl.ds(s * RPT + p * CHUNK, CHUNK)])
    pltpu.sync_copy(deg_sh.at[pl.ds(s * RPT + 4 * CHUNK, rem)],
                    ones_v.at[pl.ds(0, rem)])
    pltpu.sync_copy(ones_v.at[pl.ds(0, rem)],
                    degp_o.at[c, pl.ds(s * RPT + 4 * CHUNK, rem)])


GRP = 8
NG = CJ // GRP


@functools.partial(
    pl.kernel,
    out_type=jax.ShapeDtypeStruct((NC, NP, D), jnp.float32),
    mesh=_mesh,
    scratch_types=[
        pltpu.VMEM((2, GRP, CHUNK), jnp.int32),
        pltpu.VMEM((2, GRP, CHUNK), jnp.int32),
        pltpu.VMEM((CHUNK, D), jnp.float32),
        pltpu.VMEM((CHUNK, D), jnp.float32),
        pltpu.VMEM_SHARED((NP, D), jnp.float32),
        pltpu.SemaphoreType.DMA,
        pltpu.SemaphoreType.DMA,
        pltpu.SemaphoreType.DMA,
        pltpu.SemaphoreType.DMA,
    ],
)
def _seg_sum(u_hbm, rowp, colp, zrow, sparts_o,
             rp_slab, cp_slab, g0, g1, acc, sem0, sem1, sem_rp, sem_cp):
    c = lax.axis_index("c")
    s = lax.axis_index("s")
    wid = c * NS + s

    rem = RPT - 4 * CHUNK
    pltpu.sync_copy(zrow, g0)
    pltpu.async_copy(rowp.at[wid, pl.ds(0, GRP)], rp_slab.at[0], sem_rp)
    pltpu.async_copy(colp.at[wid, pl.ds(0, GRP)], cp_slab.at[0], sem_cp)
    for p in range(4):
        pltpu.async_copy(g0, acc.at[pl.ds(s * RPT + p * CHUNK, CHUNK)], sem1)
    pltpu.async_copy(g0.at[pl.ds(0, rem)],
                     acc.at[pl.ds(s * RPT + 4 * CHUNK, rem)], sem1)
    for p in range(4):
        pltpu.make_async_copy(
            g0, acc.at[pl.ds(s * RPT + p * CHUNK, CHUNK)], sem1).wait()
    pltpu.make_async_copy(g0.at[pl.ds(0, rem)],
                          acc.at[pl.ds(s * RPT + 4 * CHUNK, rem)], sem1).wait()
    pltpu.make_async_copy(rowp.at[wid, pl.ds(0, GRP)],
                          rp_slab.at[0], sem_rp).wait()
    pltpu.make_async_copy(colp.at[wid, pl.ds(0, GRP)],
                          cp_slab.at[0], sem_cp).wait()
    plsc.subcore_barrier()

    def _gstart(idx_row, buf, sem):
        pltpu.async_copy(u_hbm.at[idx_row[0]], buf.at[pl.ds(0, 64)], sem)
        pltpu.async_copy(u_hbm.at[idx_row[1]], buf.at[pl.ds(64, 64)], sem)

    def _gwait(idx_row, buf, sem):
        pltpu.make_async_copy(u_hbm.at[idx_row[0]],
                              buf.at[pl.ds(0, 64)], sem).wait()
        pltpu.make_async_copy(u_hbm.at[idx_row[1]],
                              buf.at[pl.ds(64, 64)], sem).wait()

    def _idx(gp, t):
        return (cp_slab.at[gp, t, pl.ds(0, 64)],
                cp_slab.at[gp, t, pl.ds(64, 64)])

    _gstart(_idx(0, 0), g0, sem0)

    def body(g, _):
        gp = lax.rem(g, 2)
        gn = lax.rem(g + 1, 2)

        @pl.when(g < NG - 1)
        def _():
            pltpu.async_copy(rowp.at[wid, pl.ds((g + 1) * GRP, GRP)],
                             rp_slab.at[gn], sem_rp)
            pltpu.async_copy(colp.at[wid, pl.ds((g + 1) * GRP, GRP)],
                             cp_slab.at[gn], sem_cp)

        for t in range(GRP):
            buf, sem = (g0, sem0) if t % 2 == 0 else (g1, sem1)
            obuf, osem = (g1, sem1) if t % 2 == 0 else (g0, sem0)
            _gwait(_idx(gp, t), buf, sem)
            if t < GRP - 1:
                _gstart(_idx(gp, t + 1), obuf, osem)
            else:
                @pl.when(g < NG - 1)
                def _():
                    pltpu.make_async_copy(
                        rowp.at[wid, pl.ds((g + 1) * GRP, GRP)],
                        rp_slab.at[gn], sem_rp).wait()
                    pltpu.make_async_copy(
                        colp.at[wid, pl.ds((g + 1) * GRP, GRP)],
                        cp_slab.at[gn], sem_cp).wait()
                    _gstart(_idx(gn, 0), obuf, osem)
            pltpu.sync_copy(buf, acc.at[rp_slab.at[gp, t]], add=True)
        return 0

    lax.fori_loop(0, NG, body, 0)

    plsc.subcore_barrier()
    for p in range(4):
        pltpu.sync_copy(deg_sh.at[pl.ds(s * RPT + p * CHUNK, CHUNK)], ones_v)
        pltpu.sync_copy(ones_v, degp_o.at[c, pl.ds(s * RPT + p * CHUNK, CHUNK)])
    pltpu.sync_copy(deg_sh.at[pl.ds(s * RPT + 4 * CHUNK, rem)],
                    ones_v.at[pl.ds(0, rem)])
    pltpu.sync_copy(ones_v.at[pl.ds(0, rem)],
                    degp_o.at[c, pl.ds(s * RPT + 4 * CHUNK, rem)])


GRP = 8
NG = CJ // GRP


@functools.partial(
    pl.kernel,
    out_type=jax.ShapeDtypeStruct((NC, NP, D), jnp.float32),
    mesh=_mesh,
    scratch_types=[
        pltpu.VMEM((2, GRP, CHUNK), jnp.int32),
        pltpu.VMEM((2, GRP, CHUNK), jnp.int32),
        pltpu.VMEM((CHUNK, D), jnp.float32),
        pltpu.VMEM((CHUNK, D), jnp.float32),
        pltpu.VMEM_SHARED((NP, D), jnp.float32),
        pltpu.SemaphoreType.DMA,
        pltpu.SemaphoreType.DMA,
        pltpu.SemaphoreType.DMA,
        pltpu.SemaphoreType.DMA,
    ],
)
def _seg_sum(u_hbm, rowp, colp, zrow, sparts_o,
             rp_slab, cp_slab, g0, g1, acc, sem0, sem1, sem_rp, sem_cp):
    c = lax.axis_index("c")
    s = lax.axis_index("s")
    wid = c * NS + s

    rem = RPT - 4 * CHUNK
    pltpu.sync_copy(zrow, g0)
    pltpu.async_copy(rowp.at[wid, pl.ds(0, GRP)], rp_slab.at[0], sem_rp)
    pltpu.async_copy(colp.at[wid, pl.ds(0, GRP)], cp_slab.at[0], sem_cp)
    for p in range(4):
        pltpu.async_copy(g0, acc.at[pl.ds(s * RPT + p * CHUNK, CHUNK)], sem1)
    pltpu.async_copy(g0.at[pl.ds(0, rem)],
                     acc.at[pl.ds(s * RPT + 4 * CHUNK, rem)], sem1)
    for p in range(4):
        pltpu.make_async_copy(
            g0, acc.at[pl.ds(s * RPT + p * CHUNK, CHUNK)], sem1).wait()
    pltpu.make_async_copy(g0.at[pl.ds(0, rem)],
                          acc.at[pl.ds(s * RPT + 4 * CHUNK, rem)], sem1).wait()
    pltpu.make_async_copy(rowp.at[wid, pl.ds(0, GRP)],
                          rp_slab.at[0], sem_rp).wait()
    pltpu.make_async_copy(colp.at[wid, pl.ds(0, GRP)],
                          cp_slab.at[0], sem_cp).wait()
    plsc.subcore_barrier()

    def _gstart(idx_row, buf, sem):
        pltpu.async_copy(u_hbm.at[idx_row[0]], buf.at[pl.ds(0, 64)], sem)
        pltpu.async_copy(u_hbm.at[idx_row[1]], buf.at[pl.ds(64, 64)], sem)

    def _gwait(idx_row, buf, sem):
        pltpu.make_async_copy(u_hbm.at[idx_row[0]],
                              buf.at[pl.ds(0, 64)], sem).wait()
        pltpu.make_async_copy(u_hbm.at[idx_row[1]],
                              buf.at[pl.ds(64, 64)], sem).wait()

    def _idx(gp, t):
        return (cp_slab.at[gp, t, pl.ds(0, 64)],
                cp_slab.at[gp, t, pl.ds(64, 64)])

    _gstart(_idx(0, 0), g0, sem0)

    def body(g, _):
        gp = lax.rem(g, 2)
        gn = lax.rem(g + 1, 2)

        @pl.when(g < NG - 1)
        def _():
            pltpu.async_copy(rowp.at[wid, pl.ds((g + 1) * GRP, GRP)],
                             rp_slab.at[gn], sem_rp)
            pltpu.async_copy(colp.at[wid, pl.ds((g + 1) * GRP, GRP)],
                             cp_slab.at[gn], sem_cp)

        for t in range(GRP):
            buf, sem = (g0, sem0) if t % 2 == 0 else (g1, sem1)
            obuf, osem = (g1, sem1) if t % 2 == 0 else (g0, sem0)
            _gwait(_idx(gp, t), buf, sem)
            if t < GRP - 1:
                _gstart(_idx(gp, t + 1), obuf, osem)
            else:
                @pl.when(g < NG - 1)
                def _():
                    pltpu.make_async_copy(
                        rowp.at[wid, pl.ds((g + 1) * GRP, GRP)],
                        rp_slab.at[gn], sem_rp).wait()
                    pltpu.make_async_copy(
                        colp.at[wid, pl.ds((g + 1) * GRP, GRP)],
                        cp_slab.at[gn], sem_cp).wait()
                    _gstart(_idx(gn, 0), obuf, osem)
            pltpu.sync_copy(buf, acc.at[rp_slab.at[gp, t]], add=True)
        return 0

    lax.fori_loop(0, NG, body, 0)

    plsc.subcore_barrier()

    def _hbm_piece(p, nrows=CHUNK):
        return sparts_o.at[c, pl.ds(s * RPT + p * CHUNK, nrows)]

    def _acc_piece(p, nrows=CHUNK):
        return acc.at[pl.ds(s * RPT + p * CHUNK, nrows)]

    bufs = [g0, g1]
    sems = [sem0, sem1]
    for p in range(4):
        b, sm = bufs[p % 2], sems[p % 2]
        if p >= 2:
            pltpu.make_async_copy(b, _hbm_piece(p - 2), sm).wait()
        pltpu.sync_copy(_acc_piece(p), b)
        pltpu.async_copy(b, _hbm_piece(p), sm)
    pltpu.make_async_copy(g0, _hbm_piece(2), sem0).wait()
    pltpu.sync_copy(_acc_piece(4, rem), g0.at[pl.ds(0, rem)])
    pltpu.async_copy(g0.at[pl.ds(0, rem)], _hbm_piece(4, rem), sem0)
    pltpu.make_async_copy(g1, _hbm_piece(3), sem1).wait()
    pltpu.make_async_copy(g0.at[pl.ds(0, rem)], _hbm_piece(4, rem),
                          sem0).wait()


def _prep_body(dp0, dp1, x, dis_o, u_o):
    deg = dp0[pl.ds(0, N), :] + dp1[pl.ds(0, N), :]
    dis = jnp.where(deg > 0, lax.rsqrt(jnp.maximum(deg, 1e-12)), 0.0)
    dis_o[...] = dis
    u_o[...] = dis * x[...]


_prep = pl.pallas_call(
    _prep_body,
    out_shape=(
        jax.ShapeDtypeStruct((N, D), jnp.float32),
        jax.ShapeDtypeStruct((N, D), jnp.float32),
    ),
)


def _layer_body(h, s0, s1, dis, wa, wb, bias, h_o, u_o, *, act):
    d = dis[...]
    t1 = -d * (s0[...] + s1[...])
    pre = (jnp.dot(h[...], wa[...], preferred_element_type=jnp.float32)
           + jnp.dot(t1, wb[...], preferred_element_type=jnp.float32)
           + bias[...])
    hn = jnp.maximum(pre, 0.0) if act else pre
    h_o[...] = hn
    u_o[...] = d * hn


def _final_body(h, s0, s1, dis, wa, wb, bias, h_o):
    t1 = -dis[...] * (s0[...] + s1[...])
    h_o[...] = (jnp.dot(h[...], wa[...], preferred_element_type=jnp.float32)
                + jnp.dot(t1, wb[...], preferred_element_type=jnp.float32)
                + bias[...])


BLK = N // 5


def _row_spec():
    return pl.BlockSpec((BLK, D), lambda i: (i, 0))


def _make_layer(final):
    body = _final_body if final else functools.partial(_layer_body, act=True)
    n_out = 1 if final else 2
    return pl.pallas_call(
        body,
        grid=(5,),
        in_specs=[_row_spec(), _row_spec(), _row_spec(), _row_spec(),
                  pl.BlockSpec((D, D), lambda i: (0, 0)),
                  pl.BlockSpec((D, D), lambda i: (0, 0)),
                  pl.BlockSpec((1, D), lambda i: (0, 0))],
        out_specs=(tuple(_row_spec() for _ in range(n_out))
                   if not final else _row_spec()),
        out_shape=(tuple(jax.ShapeDtypeStruct((N, D), jnp.float32)
                         for _ in range(n_out))
                   if not final else jax.ShapeDtypeStruct((N, D), jnp.float32)),
    )


_layer_relu = _make_layer(False)
_final = _make_layer(True)


@jax.jit
def kernel(x, edge_index, W0_0, W0_1, b0, W1_0, W1_1, b1, W2_0, W2_1, b2):
    pad = jnp.zeros((EP - E,), jnp.int32)
    row3 = jnp.concatenate([edge_index[0], pad]).reshape(NW, CJ, CHUNK)
    col3 = jnp.concatenate([edge_index[1], pad]).reshape(NW, CJ, CHUNK)

    zrow = jnp.zeros((CHUNK, D), jnp.float32)
    ones128 = jnp.ones((CHUNK, D), jnp.float32)

    rowp, colp, degp = _edge_prep(row3, col3, zrow, ones128)
    dis, u = _prep(degp[0], degp[1], x)

    h = x
    for li, (wa, wb, bias) in enumerate([(W0_0, W0_1, b0), (W1_0, W1_1, b1),
                                         (W2_0, W2_1, b2)]):
        sp = _seg_sum(u, rowp, colp, zrow)
        if li < 2:
            h, u = _layer_relu(h, sp[0], sp[1], dis, wa, wb,
                               bias.reshape(1, D))
        else:
            h = _final(h, sp[0], sp[1], dis, wa, wb, bias.reshape(1, D))
    return h


# speedup vs baseline: 1.1020x; 1.1020x over previous
import functools

import jax
import jax.numpy as jnp
from jax import lax
from jax.experimental import pallas as pl
from jax.experimental.pallas import tpu as pltpu
from jax.experimental.pallas import tpu_sc as plsc

N = 10000
E = 320000
D = 128

NC = 2
NS = 16
NW = NC * NS
CHUNK = 128
CJ = 80
ET = CJ * CHUNK
EP = NW * ET
NP = 10112
NDUM = NP - N
RPT = NP // NS

_mesh = plsc.VectorSubcoreMesh(core_axis_name="c", subcore_axis_name="s")


@functools.partial(
    pl.kernel,
    out_type=(
        jax.ShapeDtypeStruct((NW, CJ, CHUNK), jnp.int32),
        jax.ShapeDtypeStruct((NW, CJ, CHUNK), jnp.int32),
        jax.ShapeDtypeStruct((NC, NP, 8), jnp.float32),
    ),
    mesh=_mesh,
    scratch_types=[
        pltpu.VMEM((CJ, CHUNK), jnp.int32),
        pltpu.VMEM((CJ, CHUNK), jnp.int32),
        pltpu.VMEM((CHUNK, 8), jnp.float32),
        pltpu.VMEM_SHARED((NP, 8), jnp.float32),
        pltpu.SemaphoreType.DMA,
    ],
)
def _edge_prep(row3, col3, z8, ones8, rowp_o, colp_o, degp_o,
               r_buf, c_buf, ones_v, deg_sh, ssem):
    c = lax.axis_index("c")
    s = lax.axis_index("s")
    wid = c * NS + s
    rem = RPT - 4 * CHUNK

    pltpu.sync_copy(z8, ones_v)
    for p in range(4):
        pltpu.sync_copy(ones_v, deg_sh.at[pl.ds(s * RPT + p * CHUNK, CHUNK)])
    pltpu.sync_copy(ones_v.at[pl.ds(0, rem)],
                    deg_sh.at[pl.ds(s * RPT + 4 * CHUNK, rem)])
    pltpu.sync_copy(ones8, ones_v)
    pltpu.sync_copy(row3.at[wid], r_buf)
    pltpu.sync_copy(col3.at[wid], c_buf)

    base = wid * ET
    lane = lax.broadcasted_iota(jnp.int32, (16,), 0)

    def compute_row(j, _):
        for k in range(CHUNK // 16):
            r = r_buf[j, pl.ds(k * 16, 16)]
            cc = c_buf[j, pl.ds(k * 16, 16)]
            m = r == cc
            pos = base + j * CHUNK + k * 16 + lane
            dummy = N + lax.rem(pos, NDUM)
            r_buf[j, pl.ds(k * 16, 16)] = jnp.where(m, dummy, r)
            c_buf[j, pl.ds(k * 16, 16)] = jnp.where(
                m, lax.bitwise_and(pos, 8191), cc)
        return 0

    lax.fori_loop(0, CJ, compute_row, 0)
    plsc.subcore_barrier()

    def deg_row(j, _):
        pltpu.async_copy(ones_v, deg_sh.at[r_buf.at[j]], ssem, add=True)
        return 0

    lax.fori_loop(0, CJ, deg_row, 0)

    def deg_drain(j, _):
        pltpu.make_async_copy(ones_v, deg_sh.at[r_buf.at[j]], ssem).wait()
        return 0

    lax.fori_loop(0, CJ, deg_drain, 0)

    pltpu.sync_copy(r_buf, rowp_o.at[wid])
    pltpu.sync_copy(c_buf, colp_o.at[wid])
    plsc.subcore_barrier()
    for p in range(4):
        pltpu.sync_copy(deg_sh.at[pl.ds(s * RPT + p * CHUNK, CHUNK)], ones_v)
        pltpu.sync_copy(ones_v, degp_o.at[c, pl.ds(s * RPT + p * CHUNK, CHUNK)])
    pltpu.sync_copy(deg_sh.at[pl.ds(s * RPT + 4 * CHUNK, rem)],
                    ones_v.at[pl.ds(0, rem)])
    pltpu.sync_copy(ones_v.at[pl.ds(0, rem)],
                    degp_o.at[c, pl.ds(s * RPT + 4 * CHUNK, rem)])


GRP = 8
NG = CJ // GRP


@functools.partial(
    pl.kernel,
    out_type=jax.ShapeDtypeStruct((NC, NP, D), jnp.float32),
    mesh=_mesh,
    scratch_types=[
        pltpu.VMEM((2, GRP, CHUNK), jnp.int32),
        pltpu.VMEM((2, GRP, CHUNK), jnp.int32),
        pltpu.VMEM((CHUNK, D), jnp.float32),
        pltpu.VMEM((CHUNK, D), jnp.float32),
        pltpu.VMEM_SHARED((NP, D), jnp.float32),
        pltpu.SemaphoreType.DMA,
        pltpu.SemaphoreType.DMA,
        pltpu.SemaphoreType.DMA,
        pltpu.SemaphoreType.DMA,
    ],
)
def _seg_sum(u_hbm, rowp, colp, zrow, sparts_o,
             rp_slab, cp_slab, g0, g1, acc, sem0, sem1, sem_rp, sem_cp):
    c = lax.axis_index("c")
    s = lax.axis_index("s")
    wid = c * NS + s

    rem = RPT - 4 * CHUNK
    pltpu.sync_copy(zrow, g0)
    pltpu.async_copy(rowp.at[wid, pl.ds(0, GRP)], rp_slab.at[0], sem_rp)
    pltpu.async_copy(colp.at[wid, pl.ds(0, GRP)], cp_slab.at[0], sem_cp)
    for p in range(4):
        pltpu.async_copy(g0, acc.at[pl.ds(s * RPT + p * CHUNK, CHUNK)], sem1)
    pltpu.async_copy(g0.at[pl.ds(0, rem)],
                     acc.at[pl.ds(s * RPT + 4 * CHUNK, rem)], sem1)
    for p in range(4):
        pltpu.make_async_copy(
            g0, acc.at[pl.ds(s * RPT + p * CHUNK, CHUNK)], sem1).wait()
    pltpu.make_async_copy(g0.at[pl.ds(0, rem)],
                          acc.at[pl.ds(s * RPT + 4 * CHUNK, rem)], sem1).wait()
    pltpu.make_async_copy(rowp.at[wid, pl.ds(0, GRP)],
                          rp_slab.at[0], sem_rp).wait()
    pltpu.make_async_copy(colp.at[wid, pl.ds(0, GRP)],
                          cp_slab.at[0], sem_cp).wait()
    plsc.subcore_barrier()

    def _gstart(idx_row, buf, sem):
        pltpu.async_copy(u_hbm.at[idx_row[0]], buf.at[pl.ds(0, 64)], sem)
        pltpu.async_copy(u_hbm.at[idx_row[1]], buf.at[pl.ds(64, 64)], sem)

    def _gwait(idx_row, buf, sem):
        pltpu.make_async_copy(u_hbm.at[idx_row[0]],
                              buf.at[pl.ds(0, 64)], sem).wait()
        pltpu.make_async_copy(u_hbm.at[idx_row[1]],
                              buf.at[pl.ds(64, 64)], sem).wait()

    def _idx(gp, t):
        return (cp_slab.at[gp, t, pl.ds(0, 64)],
                cp_slab.at[gp, t, pl.ds(64, 64)])

    _gstart(_idx(0, 0), g0, sem0)

    def body(g, _):
        gp = lax.rem(g, 2)
        gn = lax.rem(g + 1, 2)

        @pl.when(g < NG - 1)
        def _():
            pltpu.async_copy(rowp.at[wid, pl.ds((g + 1) * GRP, GRP)],
                             rp_slab.at[gn], sem_rp)
            pltpu.async_copy(colp.at[wid, pl.ds((g + 1) * GRP, GRP)],
                             cp_slab.at[gn], sem_cp)

        for t in range(GRP):
            buf, sem = (g0, sem0) if t % 2 == 0 else (g1, sem1)
            obuf, osem = (g1, sem1) if t % 2 == 0 else (g0, sem0)
            _gwait(_idx(gp, t), buf, sem)
            if t < GRP - 1:
                _gstart(_idx(gp, t + 1), obuf, osem)
            else:
                @pl.when(g < NG - 1)
                def _():
                    pltpu.make_async_copy(
                        rowp.at[wid, pl.ds((g + 1) * GRP, GRP)],
                        rp_slab.at[gn], sem_rp).wait()
                    pltpu.make_async_copy(
                        colp.at[wid, pl.ds((g + 1) * GRP, GRP)],
                        cp_slab.at[gn], sem_cp).wait()
                    _gstart(_idx(gn, 0), obuf, osem)
            pltpu.sync_copy(buf, acc.at[rp_slab.at[gp, t]], add=True)
        return 0

    lax.fori_loop(0, NG, body, 0)

    plsc.subcore_barrier()
    for p in range(4):
        pltpu.sync_copy(deg_sh.at[pl.ds(s * RPT + p * CHUNK, CHUNK)], ones_v)
        pltpu.sync_copy(ones_v, degp_o.at[c, pl.ds(s * RPT + p * CHUNK, CHUNK)])
    pltpu.sync_copy(deg_sh.at[pl.ds(s * RPT + 4 * CHUNK, rem)],
                    ones_v.at[pl.ds(0, rem)])
    pltpu.sync_copy(ones_v.at[pl.ds(0, rem)],
                    degp_o.at[c, pl.ds(s * RPT + 4 * CHUNK, rem)])


GRP = 8
NG = CJ // GRP


@functools.partial(
    pl.kernel,
    out_type=jax.ShapeDtypeStruct((NC, NP, D), jnp.float32),
    mesh=_mesh,
    scratch_types=[
        pltpu.VMEM((2, GRP, CHUNK), jnp.int32),
        pltpu.VMEM((2, GRP, CHUNK), jnp.int32),
        pltpu.VMEM((CHUNK, D), jnp.float32),
        pltpu.VMEM((CHUNK, D), jnp.float32),
        pltpu.VMEM_SHARED((NP, D), jnp.float32),
        pltpu.SemaphoreType.DMA,
        pltpu.SemaphoreType.DMA,
        pltpu.SemaphoreType.DMA,
        pltpu.SemaphoreType.DMA,
    ],
)
def _seg_sum(u_hbm, rowp, colp, zrow, sparts_o,
             rp_slab, cp_slab, g0, g1, acc, sem0, sem1, sem_rp, sem_cp):
    c = lax.axis_index("c")
    s = lax.axis_index("s")
    wid = c * NS + s

    rem = RPT - 4 * CHUNK
    pltpu.sync_copy(zrow, g0)
    pltpu.async_copy(rowp.at[wid, pl.ds(0, GRP)], rp_slab.at[0], sem_rp)
    pltpu.async_copy(colp.at[wid, pl.ds(0, GRP)], cp_slab.at[0], sem_cp)
    for p in range(4):
        pltpu.async_copy(g0, acc.at[pl.ds(s * RPT + p * CHUNK, CHUNK)], sem1)
    pltpu.async_copy(g0.at[pl.ds(0, rem)],
                     acc.at[pl.ds(s * RPT + 4 * CHUNK, rem)], sem1)
    for p in range(4):
        pltpu.make_async_copy(
            g0, acc.at[pl.ds(s * RPT + p * CHUNK, CHUNK)], sem1).wait()
    pltpu.make_async_copy(g0.at[pl.ds(0, rem)],
                          acc.at[pl.ds(s * RPT + 4 * CHUNK, rem)], sem1).wait()
    pltpu.make_async_copy(rowp.at[wid, pl.ds(0, GRP)],
                          rp_slab.at[0], sem_rp).wait()
    pltpu.make_async_copy(colp.at[wid, pl.ds(0, GRP)],
                          cp_slab.at[0], sem_cp).wait()
    plsc.subcore_barrier()

    def _gstart(idx_row, buf, sem):
        pltpu.async_copy(u_hbm.at[idx_row[0]], buf.at[pl.ds(0, 64)], sem)
        pltpu.async_copy(u_hbm.at[idx_row[1]], buf.at[pl.ds(64, 64)], sem)

    def _gwait(idx_row, buf, sem):
        pltpu.make_async_copy(u_hbm.at[idx_row[0]],
                              buf.at[pl.ds(0, 64)], sem).wait()
        pltpu.make_async_copy(u_hbm.at[idx_row[1]],
                              buf.at[pl.ds(64, 64)], sem).wait()

    def _idx(gp, t):
        return (cp_slab.at[gp, t, pl.ds(0, 64)],
                cp_slab.at[gp, t, pl.ds(64, 64)])

    _gstart(_idx(0, 0), g0, sem0)

    def body(g, _):
        gp = lax.rem(g, 2)
        gn = lax.rem(g + 1, 2)

        @pl.when(g < NG - 1)
        def _():
            pltpu.async_copy(rowp.at[wid, pl.ds((g + 1) * GRP, GRP)],
                             rp_slab.at[gn], sem_rp)
            pltpu.async_copy(colp.at[wid, pl.ds((g + 1) * GRP, GRP)],
                             cp_slab.at[gn], sem_cp)

        for t in range(GRP):
            buf, sem = (g0, sem0) if t % 2 == 0 else (g1, sem1)
            obuf, osem = (g1, sem1) if t % 2 == 0 else (g0, sem0)
            _gwait(_idx(gp, t), buf, sem)
            if t < GRP - 1:
                _gstart(_idx(gp, t + 1), obuf, osem)
            else:
                @pl.when(g < NG - 1)
                def _():
                    pltpu.make_async_copy(
                        rowp.at[wid, pl.ds((g + 1) * GRP, GRP)],
                        rp_slab.at[gn], sem_rp).wait()
                    pltpu.make_async_copy(
                        colp.at[wid, pl.ds((g + 1) * GRP, GRP)],
                        cp_slab.at[gn], sem_cp).wait()
                    _gstart(_idx(gn, 0), obuf, osem)
            pltpu.sync_copy(buf, acc.at[rp_slab.at[gp, t]], add=True)
        return 0

    lax.fori_loop(0, NG, body, 0)

    plsc.subcore_barrier()

    def _hbm_piece(p, nrows=CHUNK):
        return sparts_o.at[c, pl.ds(s * RPT + p * CHUNK, nrows)]

    def _acc_piece(p, nrows=CHUNK):
        return acc.at[pl.ds(s * RPT + p * CHUNK, nrows)]

    bufs = [g0, g1]
    sems = [sem0, sem1]
    for p in range(4):
        b, sm = bufs[p % 2], sems[p % 2]
        if p >= 2:
            pltpu.make_async_copy(b, _hbm_piece(p - 2), sm).wait()
        pltpu.sync_copy(_acc_piece(p), b)
        pltpu.async_copy(b, _hbm_piece(p), sm)
    pltpu.make_async_copy(g0, _hbm_piece(2), sem0).wait()
    pltpu.sync_copy(_acc_piece(4, rem), g0.at[pl.ds(0, rem)])
    pltpu.async_copy(g0.at[pl.ds(0, rem)], _hbm_piece(4, rem), sem0)
    pltpu.make_async_copy(g1, _hbm_piece(3), sem1).wait()
    pltpu.make_async_copy(g0.at[pl.ds(0, rem)], _hbm_piece(4, rem),
                          sem0).wait()


def _prep_body(dp0, dp1, x, dis_o, u_o):
    deg8 = dp0[pl.ds(0, N), :] + dp1[pl.ds(0, N), :]
    dis8 = jnp.where(deg8 > 0, lax.rsqrt(jnp.maximum(deg8, 1e-12)), 0.0)
    dis = jax.lax.broadcast_in_dim(dis8[:, 0:1], (N, D), (0, 1))
    dis_o[...] = dis
    u_o[...] = dis * x[...]


_prep = pl.pallas_call(
    _prep_body,
    out_shape=(
        jax.ShapeDtypeStruct((N, D), jnp.float32),
        jax.ShapeDtypeStruct((N, D), jnp.float32),
    ),
)


def _layer_body(h, s0, s1, dis, wa, wb, bias, h_o, u_o, *, act):
    d = dis[...]
    t1 = -d * (s0[...] + s1[...])
    pre = (jnp.dot(h[...], wa[...], preferred_element_type=jnp.float32)
           + jnp.dot(t1, wb[...], preferred_element_type=jnp.float32)
           + bias[...])
    hn = jnp.maximum(pre, 0.0) if act else pre
    h_o[...] = hn
    u_o[...] = d * hn


def _final_body(h, s0, s1, dis, wa, wb, bias, h_o):
    t1 = -dis[...] * (s0[...] + s1[...])
    h_o[...] = (jnp.dot(h[...], wa[...], preferred_element_type=jnp.float32)
                + jnp.dot(t1, wb[...], preferred_element_type=jnp.float32)
                + bias[...])


BLK = N // 5


def _row_spec():
    return pl.BlockSpec((BLK, D), lambda i: (i, 0))


def _make_layer(final):
    body = _final_body if final else functools.partial(_layer_body, act=True)
    n_out = 1 if final else 2
    return pl.pallas_call(
        body,
        grid=(5,),
        in_specs=[_row_spec(), _row_spec(), _row_spec(), _row_spec(),
                  pl.BlockSpec((D, D), lambda i: (0, 0)),
                  pl.BlockSpec((D, D), lambda i: (0, 0)),
                  pl.BlockSpec((1, D), lambda i: (0, 0))],
        out_specs=(tuple(_row_spec() for _ in range(n_out))
                   if not final else _row_spec()),
        out_shape=(tuple(jax.ShapeDtypeStruct((N, D), jnp.float32)
                         for _ in range(n_out))
                   if not final else jax.ShapeDtypeStruct((N, D), jnp.float32)),
    )


_layer_relu = _make_layer(False)
_final = _make_layer(True)


@jax.jit
def kernel(x, edge_index, W0_0, W0_1, b0, W1_0, W1_1, b1, W2_0, W2_1, b2):
    pad = jnp.zeros((EP - E,), jnp.int32)
    row3 = jnp.concatenate([edge_index[0], pad]).reshape(NW, CJ, CHUNK)
    col3 = jnp.concatenate([edge_index[1], pad]).reshape(NW, CJ, CHUNK)

    zrow = jnp.zeros((CHUNK, D), jnp.float32)
    z8 = jnp.zeros((CHUNK, 8), jnp.float32)
    ones8 = jnp.ones((CHUNK, 8), jnp.float32)

    rowp, colp, degp = _edge_prep(row3, col3, z8, ones8)
    dis, u = _prep(degp[0], degp[1], x)

    h = x
    for li, (wa, wb, bias) in enumerate([(W0_0, W0_1, b0), (W1_0, W1_1, b1),
                                         (W2_0, W2_1, b2)]):
        sp = _seg_sum(u, rowp, colp, zrow)
        if li < 2:
            h, u = _layer_relu(h, sp[0], sp[1], dis, wa, wb,
                               bias.reshape(1, D))
        else:
            h = _final(h, sp[0], sp[1], dis, wa, wb, bias.reshape(1, D))
    return h
